# Initial kernel scaffold; baseline (speedup 1.0000x reference)
import jax
import jax.numpy as jnp
from jax.experimental import pallas as pl


def kernel(x_lm, x, edge_index, edge_type, edge_attr, triplet_embedding, neg_tail, W1, rel_emb, W2, k):
    raise NotImplementedError("write your pallas kernel here")


# speedup vs baseline: 1.3448x; 1.3448x over previous
import functools

import jax
import jax.numpy as jnp
from jax import lax
from jax.experimental import pallas as pl
from jax.experimental.pallas import tpu as pltpu
from jax.experimental.pallas import tpu_sc as plsc

N = 10000
E = 160000
D = 256
DE = 16
R = 50
B = 16
NEG = 8
K = 10
RP = 64

NC = 2
NSC = 16
NW = NC * NSC

CH = 128
PER_W = 5120
E2 = NW * PER_W
EB = 1280
NB = E2 // EB
NBV = E // EB

F32 = jnp.float32
BF16 = jnp.bfloat16
I32 = jnp.int32
HI = lax.Precision.HIGHEST


def _gather_add_all(xa, xb, src_pad, dst_pad):
    mesh = plsc.VectorSubcoreMesh(core_axis_name="c", subcore_axis_name="s")

    @functools.partial(
        pl.kernel,
        out_type=jax.ShapeDtypeStruct((E2, D), F32),
        mesh=mesh,
        scratch_types=[
            pltpu.VMEM((CH,), I32),
            pltpu.VMEM((CH,), I32),
            pltpu.VMEM((CH, D), F32),
            pltpu.VMEM((CH, D), F32),
            pltpu.SemaphoreType.DMA,
            pltpu.SemaphoreType.DMA,
        ],
    )
    def k(xa_h, xb_h, src_h, dst_h, out_h, isrc, idst, ra, rb, sem1, sem2):
        wid = lax.axis_index("s") * NC + lax.axis_index("c")
        base = wid * PER_W

        def chunk(c, carry):
            off = pl.multiple_of(base + c * CH, CH)
            pltpu.sync_copy(src_h.at[pl.ds(off, CH)], isrc)
            pltpu.sync_copy(dst_h.at[pl.ds(off, CH)], idst)
            cp1 = pltpu.async_copy(xa_h.at[isrc], ra, sem1)
            cp2 = pltpu.async_copy(xb_h.at[idst], rb, sem2)
            cp1.wait()
            cp2.wait()

            def addrow(r, carry2):
                for c16 in range(D // 16):
                    sl = pl.ds(c16 * 16, 16)
                    ra[r, sl] = ra[r, sl] + rb[r, sl]
                return carry2

            lax.fori_loop(0, CH, addrow, 0, unroll=False)
            pltpu.sync_copy(ra, out_h.at[pl.ds(off, CH)])
            return carry

        lax.fori_loop(0, PER_W // CH, chunk, 0, unroll=False)

    return k(xa, xb, src_pad, dst_pad)


def _score_body(hp_ref, base_ref, et_ref, relp_ref, xlm_ref, w2_ref,
                tki_ref, s_scr):
    i = pl.program_id(0)
    et = et_ref[0]
    oh = (lax.broadcasted_iota(I32, (RP, EB), 0) == et).astype(F32)
    relg = lax.dot_general(oh, relp_ref[...], (((0,), (0,)), ((), ())),
                           preferred_element_type=F32, precision=HI)
    g = jnp.maximum((hp_ref[...] + base_ref[...]) + relg, 0.0)
    kg = jnp.dot(g.astype(BF16), w2_ref[...].astype(BF16),
                 preferred_element_type=F32)
    kgbf = kg.astype(BF16)
    xlm = xlm_ref[...]
    s1 = lax.dot_general(xlm[:, :128].astype(BF16), kgbf[:, :128],
                         (((1,), (1,)), ((), ())), preferred_element_type=F32)
    s2 = lax.dot_general(xlm[:, 128:].astype(BF16), kgbf[:, 128:],
                         (((1,), (1,)), ((), ())), preferred_element_type=F32)
    sT = s1 + s2
    col = lax.broadcasted_iota(I32, (B, EB), 1) + i * EB
    sT = jnp.where(col < E, sT, F32(-3e38))
    s_scr[:, pl.ds(i * EB, EB)] = sT

    @pl.when(i == NB - 1)
    def _():
        s = s_scr[...]
        iota = lax.broadcasted_iota(I32, (B, E2), 1)
        out_iota = lax.broadcasted_iota(I32, (B, 128), 1)
        tk = jnp.zeros((B, 128), I32)
        for j in range(K):
            m = jnp.max(s, axis=1, keepdims=True)
            idx = jnp.min(jnp.where(s == m, iota, I32(E2)), axis=1,
                          keepdims=True)
            tk = jnp.where(out_iota == j, idx, tk)
            s = jnp.where(iota == idx, F32(-3e38), s)
        tki_ref[...] = tk


def _score_topk(h_pre, base, etype3, relp, x_lm, W2):
    clamp = lambda i: jnp.minimum(i, NBV - 1)
    return pl.pallas_call(
        _score_body,
        grid=(NB,),
        in_specs=[
            pl.BlockSpec((EB, D), lambda i: (i, 0)),
            pl.BlockSpec((EB, D), lambda i: (clamp(i), 0)),
            pl.BlockSpec((1, 1, EB), lambda i: (clamp(i), 0, 0)),
            pl.BlockSpec((RP, D), lambda i: (0, 0)),
            pl.BlockSpec((B, D), lambda i: (0, 0)),
            pl.BlockSpec((D, D), lambda i: (0, 0)),
        ],
        out_specs=pl.BlockSpec((B, 128), lambda i: (0, 0)),
        out_shape=jax.ShapeDtypeStruct((B, 128), I32),
        scratch_shapes=[pltpu.VMEM((B, E2), F32)],
    )(h_pre, base, etype3, relp, x_lm, W2)


def _gather_tail(xa, xb, h_pre, trip, neg_src, neg_tail, topk_pad):
    mesh = plsc.VectorSubcoreMesh(core_axis_name="c", subcore_axis_name="s")
    MN = B * K * NEG
    MC = 256
    nw_n = MN // NW
    nw_c = MC // NW

    @functools.partial(
        pl.kernel,
        out_type=(
            jax.ShapeDtypeStruct((MN, D), F32),
            jax.ShapeDtypeStruct((MN, D), F32),
            jax.ShapeDtypeStruct((MC, D), F32),
            jax.ShapeDtypeStruct((MC, D), F32),
        ),
        mesh=mesh,
        scratch_types=[
            pltpu.VMEM((nw_n,), I32),
            pltpu.VMEM((nw_n,), I32),
            pltpu.VMEM((nw_c,), I32),
            pltpu.VMEM((nw_n, D), F32),
            pltpu.VMEM((nw_n, D), F32),
            pltpu.VMEM((nw_c, D), F32),
            pltpu.VMEM((nw_c, D), F32),
            pltpu.SemaphoreType.DMA,
            pltpu.SemaphoreType.DMA,
            pltpu.SemaphoreType.DMA,
            pltpu.SemaphoreType.DMA,
        ],
    )
    def k(xa_h, xb_h, hp_h, tr_h, ns_h, nt_h, tk_h,
          o1, o2, o3, o4, i1, i2, i3, r1, r2, r3, r4, s1, s2, s3, s4):
        wid = lax.axis_index("s") * NC + lax.axis_index("c")
        pltpu.sync_copy(ns_h.at[pl.ds(wid * nw_n, nw_n)], i1)
        pltpu.sync_copy(nt_h.at[pl.ds(wid * nw_n, nw_n)], i2)
        pltpu.sync_copy(tk_h.at[pl.ds(wid * nw_c, nw_c)], i3)
        c1 = pltpu.async_copy(xa_h.at[i1], r1, s1)
        c2 = pltpu.async_copy(xb_h.at[i2], r2, s2)
        c3 = pltpu.async_copy(hp_h.at[i3], r3, s3)
        c4 = pltpu.async_copy(tr_h.at[i3], r4, s4)
        c1.wait()
        c2.wait()
        c3.wait()
        c4.wait()
        pltpu.sync_copy(r1, o1.at[pl.ds(wid * nw_n, nw_n)])
        pltpu.sync_copy(r2, o2.at[pl.ds(wid * nw_n, nw_n)])
        pltpu.sync_copy(r3, o3.at[pl.ds(wid * nw_c, nw_c)])
        pltpu.sync_copy(r4, o4.at[pl.ds(wid * nw_c, nw_c)])

    return k(xa, xb, h_pre, trip, neg_src, neg_tail, topk_pad)


def _tail_body(hc_ref, ca_ref, ct_ref, gxa_ref, gxb_ref, na_ref, nt_ref,
               w1c_ref, relp_ref, w2_ref, xlc_ref, xln_ref,
               cz_ref, nz_ref, pos_ref, neg_ref):
    w1c = w1c_ref[...]
    relp = relp_ref[...]
    w2 = w2_ref[...]
    ohc = (lax.broadcasted_iota(I32, (256, RP), 1) == ct_ref[...]).astype(F32)
    bc = (jnp.dot(ca_ref[...], w1c, preferred_element_type=F32, precision=HI)
          + jnp.dot(ohc, relp, preferred_element_type=F32, precision=HI))
    cz = jnp.dot(jnp.maximum(hc_ref[...] + bc, 0.0), w2,
                 preferred_element_type=F32, precision=HI)
    cz_ref[...] = cz
    pos = jnp.sum(cz * xlc_ref[...], axis=1, keepdims=True)
    pos_ref[...] = jnp.broadcast_to(pos, (256, 128))
    ohn = (lax.broadcasted_iota(I32, (B * K * NEG, RP), 1)
           == nt_ref[...]).astype(F32)
    bn = (jnp.dot(na_ref[...], w1c, preferred_element_type=F32, precision=HI)
          + jnp.dot(ohn, relp, preferred_element_type=F32, precision=HI))
    hn = gxa_ref[...] + gxb_ref[...] + bn
    nz = jnp.dot(jnp.maximum(hn, 0.0), w2, preferred_element_type=F32,
                 precision=HI)
    nz_ref[...] = nz
    neg = jnp.sum(nz * xln_ref[...], axis=1, keepdims=True)
    neg_ref[...] = jnp.broadcast_to(neg, (B * K * NEG, 128))


def _tail(h_cand, cand_attr, cand_type, gxa, gxb, neg_attr, neg_type,
          W1c, relp, W2, xl_rep_c, xl_rep_n):
    MN = B * K * NEG
    return pl.pallas_call(
        _tail_body,
        out_shape=[
            jax.ShapeDtypeStruct((256, D), F32),
            jax.ShapeDtypeStruct((MN, D), F32),
            jax.ShapeDtypeStruct((256, 128), F32),
            jax.ShapeDtypeStruct((MN, 128), F32),
        ],
    )(h_cand, cand_attr, cand_type, gxa, gxb, neg_attr, neg_type,
      W1c, relp, W2, xl_rep_c, xl_rep_n)


def kernel(x_lm, x, edge_index, edge_type, edge_attr, triplet_embedding,
           neg_tail, W1, rel_emb, W2, k):
    del k
    src = edge_index[0]
    dst = edge_index[1]

    xa = x @ W1[:D]
    xb = x @ W1[D:2 * D]
    base = edge_attr @ W1[2 * D:]
    pad_e = E2 - E
    src_pad = jnp.concatenate([src, jnp.zeros((pad_e,), I32)])
    dst_pad = jnp.concatenate([dst, jnp.zeros((pad_e,), I32)])
    etype3 = edge_type.reshape(NBV, 1, EB)
    relp = jnp.concatenate([rel_emb, jnp.zeros((RP - R, D), F32)])
    W1c = W1[2 * D:, :]

    h_pre = _gather_add_all(xa, xb, src_pad, dst_pad)

    tki = _score_topk(h_pre, base, etype3, relp, x_lm, W2)
    topk_idx = tki[:, :K]

    flat_tk = topk_idx.reshape(B * K)
    topk_pad = jnp.concatenate([flat_tk, jnp.zeros((256 - B * K,), I32)])
    cand_src = jnp.take(src, flat_tk, axis=0)
    cand_type = jnp.take(edge_type, flat_tk, axis=0)
    cand_attr = jnp.take(edge_attr, flat_tk, axis=0)

    neg_src = jnp.repeat(cand_src.reshape(B, K), NEG, axis=1).reshape(-1)
    neg_type_f = jnp.repeat(cand_type.reshape(B, K), NEG, axis=1).reshape(-1)
    neg_attr = jnp.repeat(cand_attr.reshape(B, K, DE), NEG, axis=1
                          ).reshape(-1, DE)
    neg_tail_f = neg_tail.reshape(-1)

    gxa, gxb, h_cand, trip_cand = _gather_tail(
        xa, xb, h_pre, triplet_embedding, neg_src, neg_tail_f, topk_pad)

    cand_attr_p = jnp.concatenate(
        [cand_attr, jnp.zeros((256 - B * K, DE), F32)])
    cand_type_p = jnp.concatenate(
        [cand_type, jnp.zeros((256 - B * K,), I32)]).reshape(256, 1)
    xl_rep_c = jnp.concatenate(
        [jnp.repeat(x_lm, K, axis=0), jnp.zeros((256 - B * K, D), F32)])
    xl_rep_n = jnp.repeat(x_lm, K * NEG, axis=0)

    cz, nz, pos, neg = _tail(
        h_cand, cand_attr_p, cand_type_p, gxa, gxb,
        neg_attr, neg_type_f.reshape(B * K * NEG, 1), W1c, relp, W2,
        xl_rep_c, xl_rep_n)

    candidates_z = cz[:B * K].reshape(B, K, D)
    negatives_z = nz.reshape(B, K * NEG, D)
    candidates_lm_z = trip_cand[:B * K].reshape(B, K, D)
    pos_scores = pos[:B * K, 0].reshape(B, K)
    neg_scores = neg[:, 0].reshape(B, K * NEG)

    return (x_lm, pos_scores, neg_scores, candidates_z, candidates_lm_z,
            negatives_z)


# speedup vs baseline: 1.5568x; 1.1576x over previous
import functools

import jax
import jax.numpy as jnp
from jax import lax
from jax.experimental import pallas as pl
from jax.experimental.pallas import tpu as pltpu
from jax.experimental.pallas import tpu_sc as plsc

N = 10000
E = 160000
D = 256
DE = 16
R = 50
B = 16
NEG = 8
K = 10
RP = 64

NC = 2
NSC = 16
NW = NC * NSC

CH = 64
PER_W = 5120
E2 = NW * PER_W
EB = 1280
NB = E2 // EB
NBV = E // EB

F32 = jnp.float32
BF16 = jnp.bfloat16
I32 = jnp.int32
HI = lax.Precision.HIGHEST


def _gather_add_all(xa, xb, src_pad, dst_pad):
    mesh = plsc.VectorSubcoreMesh(core_axis_name="c", subcore_axis_name="s")

    NCH = PER_W // CH

    @functools.partial(
        pl.kernel,
        out_type=jax.ShapeDtypeStruct((E2, D), F32),
        mesh=mesh,
        scratch_types=[
            pltpu.VMEM((CH,), I32),
            pltpu.VMEM((CH,), I32),
            pltpu.VMEM((CH, D), F32),
            pltpu.VMEM((CH, D), F32),
            pltpu.VMEM((CH,), I32),
            pltpu.VMEM((CH,), I32),
            pltpu.VMEM((CH, D), F32),
            pltpu.VMEM((CH, D), F32),
            pltpu.SemaphoreType.DMA,
            pltpu.SemaphoreType.DMA,
            pltpu.SemaphoreType.DMA,
            pltpu.SemaphoreType.DMA,
        ],
    )
    def k(xa_h, xb_h, src_h, dst_h, out_h,
          isrc0, idst0, ra0, rb0, isrc1, idst1, ra1, rb1,
          sa0, sb0, sa1, sb1):
        wid = lax.axis_index("s") * NC + lax.axis_index("c")
        base = wid * PER_W
        bufs = ((isrc0, idst0, ra0, rb0, sa0, sb0),
                (isrc1, idst1, ra1, rb1, sa1, sb1))

        def fire(c, buf):
            isrc, idst, ra, rb, sa, sb = buf
            off = pl.multiple_of(base + c * CH, CH)
            pltpu.sync_copy(src_h.at[pl.ds(off, CH)], isrc)
            pltpu.sync_copy(dst_h.at[pl.ds(off, CH)], idst)
            pltpu.async_copy(xa_h.at[isrc], ra, sa)
            pltpu.async_copy(xb_h.at[idst], rb, sb)

        def process(c, buf):
            isrc, idst, ra, rb, sa, sb = buf
            off = pl.multiple_of(base + c * CH, CH)
            pltpu.make_async_copy(xa_h.at[isrc], ra, sa).wait()
            pltpu.make_async_copy(xb_h.at[idst], rb, sb).wait()

            def addrow(r, carry2):
                for c16 in range(D // 16):
                    sl = pl.ds(c16 * 16, 16)
                    ra[r, sl] = ra[r, sl] + rb[r, sl]
                return carry2

            lax.fori_loop(0, CH, addrow, 0, unroll=4)
            pltpu.sync_copy(ra, out_h.at[pl.ds(off, CH)])

        fire(0, bufs[0])

        def pair(gp, carry):
            g0 = gp * 2
            fire(g0 + 1, bufs[1])
            process(g0, bufs[0])

            @pl.when(gp < NCH // 2 - 1)
            def _():
                fire(g0 + 2, bufs[0])

            process(g0 + 1, bufs[1])
            return carry

        lax.fori_loop(0, NCH // 2, pair, 0, unroll=False)

    return k(xa, xb, src_pad, dst_pad)


def _score_body(hp_ref, base_ref, et_ref, relp_ref, xlm_ref, w2_ref,
                tki_ref, s_scr):
    i = pl.program_id(0)
    et = et_ref[0]
    oh = (lax.broadcasted_iota(I32, (RP, EB), 0) == et).astype(F32)
    relg = lax.dot_general(oh, relp_ref[...], (((0,), (0,)), ((), ())),
                           preferred_element_type=F32, precision=HI)
    g = jnp.maximum((hp_ref[...] + base_ref[...]) + relg, 0.0)
    kg = jnp.dot(g.astype(BF16), w2_ref[...].astype(BF16),
                 preferred_element_type=F32)
    kgbf = kg.astype(BF16)
    xlm = xlm_ref[...]
    s1 = lax.dot_general(xlm[:, :128].astype(BF16), kgbf[:, :128],
                         (((1,), (1,)), ((), ())), preferred_element_type=F32)
    s2 = lax.dot_general(xlm[:, 128:].astype(BF16), kgbf[:, 128:],
                         (((1,), (1,)), ((), ())), preferred_element_type=F32)
    sT = s1 + s2
    col = lax.broadcasted_iota(I32, (B, EB), 1) + i * EB
    sT = jnp.where(col < E, sT, F32(-3e38))
    s_scr[:, pl.ds(i * EB, EB)] = sT

    @pl.when(i == NB - 1)
    def _():
        s = s_scr[...]
        iota = lax.broadcasted_iota(I32, (B, E2), 1)
        out_iota = lax.broadcasted_iota(I32, (B, 128), 1)
        tk = jnp.zeros((B, 128), I32)
        for j in range(K):
            m = jnp.max(s, axis=1, keepdims=True)
            idx = jnp.min(jnp.where(s == m, iota, I32(E2)), axis=1,
                          keepdims=True)
            tk = jnp.where(out_iota == j, idx, tk)
            s = jnp.where(iota == idx, F32(-3e38), s)
        tki_ref[...] = tk


def _score_topk(h_pre, base, etype3, relp, x_lm, W2):
    clamp = lambda i: jnp.minimum(i, NBV - 1)
    return pl.pallas_call(
        _score_body,
        grid=(NB,),
        in_specs=[
            pl.BlockSpec((EB, D), lambda i: (i, 0)),
            pl.BlockSpec((EB, D), lambda i: (clamp(i), 0)),
            pl.BlockSpec((1, 1, EB), lambda i: (clamp(i), 0, 0)),
            pl.BlockSpec((RP, D), lambda i: (0, 0)),
            pl.BlockSpec((B, D), lambda i: (0, 0)),
            pl.BlockSpec((D, D), lambda i: (0, 0)),
        ],
        out_specs=pl.BlockSpec((B, 128), lambda i: (0, 0)),
        out_shape=jax.ShapeDtypeStruct((B, 128), I32),
        scratch_shapes=[pltpu.VMEM((B, E2), F32)],
    )(h_pre, base, etype3, relp, x_lm, W2)


def _gather_tail(xa, xb, h_pre, trip, neg_src, neg_tail, topk_pad):
    mesh = plsc.VectorSubcoreMesh(core_axis_name="c", subcore_axis_name="s")
    MN = B * K * NEG
    MC = 256
    nw_n = MN // NW
    nw_c = MC // NW

    @functools.partial(
        pl.kernel,
        out_type=(
            jax.ShapeDtypeStruct((MN, D), F32),
            jax.ShapeDtypeStruct((MN, D), F32),
            jax.ShapeDtypeStruct((MC, D), F32),
            jax.ShapeDtypeStruct((MC, D), F32),
        ),
        mesh=mesh,
        scratch_types=[
            pltpu.VMEM((nw_n,), I32),
            pltpu.VMEM((nw_n,), I32),
            pltpu.VMEM((nw_c,), I32),
            pltpu.VMEM((nw_n, D), F32),
            pltpu.VMEM((nw_n, D), F32),
            pltpu.VMEM((nw_c, D), F32),
            pltpu.VMEM((nw_c, D), F32),
            pltpu.SemaphoreType.DMA,
            pltpu.SemaphoreType.DMA,
            pltpu.SemaphoreType.DMA,
            pltpu.SemaphoreType.DMA,
        ],
    )
    def k(xa_h, xb_h, hp_h, tr_h, ns_h, nt_h, tk_h,
          o1, o2, o3, o4, i1, i2, i3, r1, r2, r3, r4, s1, s2, s3, s4):
        wid = lax.axis_index("s") * NC + lax.axis_index("c")
        pltpu.sync_copy(ns_h.at[pl.ds(wid * nw_n, nw_n)], i1)
        pltpu.sync_copy(nt_h.at[pl.ds(wid * nw_n, nw_n)], i2)
        pltpu.sync_copy(tk_h.at[pl.ds(wid * nw_c, nw_c)], i3)
        c1 = pltpu.async_copy(xa_h.at[i1], r1, s1)
        c2 = pltpu.async_copy(xb_h.at[i2], r2, s2)
        c3 = pltpu.async_copy(hp_h.at[i3], r3, s3)
        c4 = pltpu.async_copy(tr_h.at[i3], r4, s4)
        c1.wait()
        c2.wait()
        c3.wait()
        c4.wait()
        pltpu.sync_copy(r1, o1.at[pl.ds(wid * nw_n, nw_n)])
        pltpu.sync_copy(r2, o2.at[pl.ds(wid * nw_n, nw_n)])
        pltpu.sync_copy(r3, o3.at[pl.ds(wid * nw_c, nw_c)])
        pltpu.sync_copy(r4, o4.at[pl.ds(wid * nw_c, nw_c)])

    return k(xa, xb, h_pre, trip, neg_src, neg_tail, topk_pad)


def _tail_body(hc_ref, ca_ref, ct_ref, gxa_ref, gxb_ref, na_ref, nt_ref,
               w1c_ref, relp_ref, w2_ref, xlc_ref, xln_ref,
               cz_ref, nz_ref, pos_ref, neg_ref):
    w1c = w1c_ref[...]
    relp = relp_ref[...]
    w2 = w2_ref[...]
    ohc = (lax.broadcasted_iota(I32, (256, RP), 1) == ct_ref[...]).astype(F32)
    bc = (jnp.dot(ca_ref[...], w1c, preferred_element_type=F32, precision=HI)
          + jnp.dot(ohc, relp, preferred_element_type=F32, precision=HI))
    cz = jnp.dot(jnp.maximum(hc_ref[...] + bc, 0.0), w2,
                 preferred_element_type=F32, precision=HI)
    cz_ref[...] = cz
    pos = jnp.sum(cz * xlc_ref[...], axis=1, keepdims=True)
    pos_ref[...] = jnp.broadcast_to(pos, (256, 128))
    ohn = (lax.broadcasted_iota(I32, (B * K * NEG, RP), 1)
           == nt_ref[...]).astype(F32)
    bn = (jnp.dot(na_ref[...], w1c, preferred_element_type=F32, precision=HI)
          + jnp.dot(ohn, relp, preferred_element_type=F32, precision=HI))
    hn = gxa_ref[...] + gxb_ref[...] + bn
    nz = jnp.dot(jnp.maximum(hn, 0.0), w2, preferred_element_type=F32,
                 precision=HI)
    nz_ref[...] = nz
    neg = jnp.sum(nz * xln_ref[...], axis=1, keepdims=True)
    neg_ref[...] = jnp.broadcast_to(neg, (B * K * NEG, 128))


def _tail(h_cand, cand_attr, cand_type, gxa, gxb, neg_attr, neg_type,
          W1c, relp, W2, xl_rep_c, xl_rep_n):
    MN = B * K * NEG
    return pl.pallas_call(
        _tail_body,
        out_shape=[
            jax.ShapeDtypeStruct((256, D), F32),
            jax.ShapeDtypeStruct((MN, D), F32),
            jax.ShapeDtypeStruct((256, 128), F32),
            jax.ShapeDtypeStruct((MN, 128), F32),
        ],
    )(h_cand, cand_attr, cand_type, gxa, gxb, neg_attr, neg_type,
      W1c, relp, W2, xl_rep_c, xl_rep_n)


def kernel(x_lm, x, edge_index, edge_type, edge_attr, triplet_embedding,
           neg_tail, W1, rel_emb, W2, k):
    del k
    src = edge_index[0]
    dst = edge_index[1]

    xa = x @ W1[:D]
    xb = x @ W1[D:2 * D]
    base = edge_attr @ W1[2 * D:]
    pad_e = E2 - E
    src_pad = jnp.concatenate([src, jnp.zeros((pad_e,), I32)])
    dst_pad = jnp.concatenate([dst, jnp.zeros((pad_e,), I32)])
    etype3 = edge_type.reshape(NBV, 1, EB)
    relp = jnp.concatenate([rel_emb, jnp.zeros((RP - R, D), F32)])
    W1c = W1[2 * D:, :]

    h_pre = _gather_add_all(xa, xb, src_pad, dst_pad)

    tki = _score_topk(h_pre, base, etype3, relp, x_lm, W2)
    topk_idx = tki[:, :K]

    flat_tk = topk_idx.reshape(B * K)
    topk_pad = jnp.concatenate([flat_tk, jnp.zeros((256 - B * K,), I32)])
    cand_src = jnp.take(src, flat_tk, axis=0)
    cand_type = jnp.take(edge_type, flat_tk, axis=0)
    cand_attr = jnp.take(edge_attr, flat_tk, axis=0)

    neg_src = jnp.repeat(cand_src.reshape(B, K), NEG, axis=1).reshape(-1)
    neg_type_f = jnp.repeat(cand_type.reshape(B, K), NEG, axis=1).reshape(-1)
    neg_attr = jnp.repeat(cand_attr.reshape(B, K, DE), NEG, axis=1
                          ).reshape(-1, DE)
    neg_tail_f = neg_tail.reshape(-1)

    gxa, gxb, h_cand, trip_cand = _gather_tail(
        xa, xb, h_pre, triplet_embedding, neg_src, neg_tail_f, topk_pad)

    cand_attr_p = jnp.concatenate(
        [cand_attr, jnp.zeros((256 - B * K, DE), F32)])
    cand_type_p = jnp.concatenate(
        [cand_type, jnp.zeros((256 - B * K,), I32)]).reshape(256, 1)
    xl_rep_c = jnp.concatenate(
        [jnp.repeat(x_lm, K, axis=0), jnp.zeros((256 - B * K, D), F32)])
    xl_rep_n = jnp.repeat(x_lm, K * NEG, axis=0)

    cz, nz, pos, neg = _tail(
        h_cand, cand_attr_p, cand_type_p, gxa, gxb,
        neg_attr, neg_type_f.reshape(B * K * NEG, 1), W1c, relp, W2,
        xl_rep_c, xl_rep_n)

    candidates_z = cz[:B * K].reshape(B, K, D)
    negatives_z = nz.reshape(B, K * NEG, D)
    candidates_lm_z = trip_cand[:B * K].reshape(B, K, D)
    pos_scores = pos[:B * K, 0].reshape(B, K)
    neg_scores = neg[:, 0].reshape(B, K * NEG)

    return (x_lm, pos_scores, neg_scores, candidates_z, candidates_lm_z,
            negatives_z)


# speedup vs baseline: 1.5571x; 1.0002x over previous
import functools

import jax
import jax.numpy as jnp
from jax import lax
from jax.experimental import pallas as pl
from jax.experimental.pallas import tpu as pltpu
from jax.experimental.pallas import tpu_sc as plsc

N = 10000
E = 160000
D = 256
DE = 16
R = 50
B = 16
NEG = 8
K = 10
RP = 64

NC = 2
NSC = 16
NW = NC * NSC

CH = 80
PER_W = 5120
E2 = NW * PER_W
EB = 1280
NB = E2 // EB
NBV = E // EB

F32 = jnp.float32
BF16 = jnp.bfloat16
I32 = jnp.int32
HI = lax.Precision.HIGHEST


def _gather_add_all(xa, xb, src_pad, dst_pad):
    mesh = plsc.VectorSubcoreMesh(core_axis_name="c", subcore_axis_name="s")

    NCH = PER_W // CH

    @functools.partial(
        pl.kernel,
        out_type=jax.ShapeDtypeStruct((E2, D), F32),
        mesh=mesh,
        scratch_types=[
            pltpu.VMEM((CH,), I32),
            pltpu.VMEM((CH,), I32),
            pltpu.VMEM((CH, D), F32),
            pltpu.VMEM((CH, D), F32),
            pltpu.VMEM((CH,), I32),
            pltpu.VMEM((CH,), I32),
            pltpu.VMEM((CH, D), F32),
            pltpu.VMEM((CH, D), F32),
            pltpu.SemaphoreType.DMA,
            pltpu.SemaphoreType.DMA,
            pltpu.SemaphoreType.DMA,
            pltpu.SemaphoreType.DMA,
        ],
    )
    def k(xa_h, xb_h, src_h, dst_h, out_h,
          isrc0, idst0, ra0, rb0, isrc1, idst1, ra1, rb1,
          sa0, sb0, sa1, sb1):
        wid = lax.axis_index("s") * NC + lax.axis_index("c")
        base = wid * PER_W
        bufs = ((isrc0, idst0, ra0, rb0, sa0, sb0),
                (isrc1, idst1, ra1, rb1, sa1, sb1))

        def fire(c, buf):
            isrc, idst, ra, rb, sa, sb = buf
            off = pl.multiple_of(base + c * CH, CH)
            pltpu.sync_copy(src_h.at[pl.ds(off, CH)], isrc)
            pltpu.sync_copy(dst_h.at[pl.ds(off, CH)], idst)
            pltpu.async_copy(xa_h.at[isrc], ra, sa)
            pltpu.async_copy(xb_h.at[idst], rb, sb)

        def process(c, buf):
            isrc, idst, ra, rb, sa, sb = buf
            off = pl.multiple_of(base + c * CH, CH)
            pltpu.make_async_copy(xa_h.at[isrc], ra, sa).wait()
            pltpu.make_async_copy(xb_h.at[idst], rb, sb).wait()

            def addrow(r, carry2):
                for c16 in range(D // 16):
                    sl = pl.ds(c16 * 16, 16)
                    ra[r, sl] = ra[r, sl] + rb[r, sl]
                return carry2

            lax.fori_loop(0, CH, addrow, 0, unroll=4)
            pltpu.sync_copy(ra, out_h.at[pl.ds(off, CH)])

        fire(0, bufs[0])

        def pair(gp, carry):
            g0 = gp * 2
            fire(g0 + 1, bufs[1])
            process(g0, bufs[0])

            @pl.when(gp < NCH // 2 - 1)
            def _():
                fire(g0 + 2, bufs[0])

            process(g0 + 1, bufs[1])
            return carry

        lax.fori_loop(0, NCH // 2, pair, 0, unroll=False)

    return k(xa, xb, src_pad, dst_pad)


def _score_body(hp_ref, base_ref, et_ref, relp_ref, xlm_ref, w2_ref,
                tki_ref, s_scr):
    i = pl.program_id(0)
    et = et_ref[0]
    oh = (lax.broadcasted_iota(I32, (RP, EB), 0) == et).astype(F32)
    relg = lax.dot_general(oh, relp_ref[...], (((0,), (0,)), ((), ())),
                           preferred_element_type=F32, precision=HI)
    g = jnp.maximum((hp_ref[...] + base_ref[...]) + relg, 0.0)
    kg = jnp.dot(g.astype(BF16), w2_ref[...].astype(BF16),
                 preferred_element_type=F32)
    kgbf = kg.astype(BF16)
    xlm = xlm_ref[...]
    s1 = lax.dot_general(xlm[:, :128].astype(BF16), kgbf[:, :128],
                         (((1,), (1,)), ((), ())), preferred_element_type=F32)
    s2 = lax.dot_general(xlm[:, 128:].astype(BF16), kgbf[:, 128:],
                         (((1,), (1,)), ((), ())), preferred_element_type=F32)
    sT = s1 + s2
    col = lax.broadcasted_iota(I32, (B, EB), 1) + i * EB
    sT = jnp.where(col < E, sT, F32(-3e38))
    s_scr[:, pl.ds(i * EB, EB)] = sT

    @pl.when(i == NB - 1)
    def _():
        s = s_scr[...]
        iota = lax.broadcasted_iota(I32, (B, E2), 1)
        out_iota = lax.broadcasted_iota(I32, (B, 128), 1)
        tk = jnp.zeros((B, 128), I32)
        for j in range(K):
            m = jnp.max(s, axis=1, keepdims=True)
            idx = jnp.min(jnp.where(s == m, iota, I32(E2)), axis=1,
                          keepdims=True)
            tk = jnp.where(out_iota == j, idx, tk)
            s = jnp.where(iota == idx, F32(-3e38), s)
        tki_ref[...] = tk


def _score_topk(h_pre, base, etype3, relp, x_lm, W2):
    clamp = lambda i: jnp.minimum(i, NBV - 1)
    return pl.pallas_call(
        _score_body,
        grid=(NB,),
        in_specs=[
            pl.BlockSpec((EB, D), lambda i: (i, 0)),
            pl.BlockSpec((EB, D), lambda i: (clamp(i), 0)),
            pl.BlockSpec((1, 1, EB), lambda i: (clamp(i), 0, 0)),
            pl.BlockSpec((RP, D), lambda i: (0, 0)),
            pl.BlockSpec((B, D), lambda i: (0, 0)),
            pl.BlockSpec((D, D), lambda i: (0, 0)),
        ],
        out_specs=pl.BlockSpec((B, 128), lambda i: (0, 0)),
        out_shape=jax.ShapeDtypeStruct((B, 128), I32),
        scratch_shapes=[pltpu.VMEM((B, E2), F32)],
    )(h_pre, base, etype3, relp, x_lm, W2)


def _gather_tail(xa, xb, h_pre, trip, neg_src, neg_tail, topk_pad):
    mesh = plsc.VectorSubcoreMesh(core_axis_name="c", subcore_axis_name="s")
    MN = B * K * NEG
    MC = 256
    nw_n = MN // NW
    nw_c = MC // NW

    @functools.partial(
        pl.kernel,
        out_type=(
            jax.ShapeDtypeStruct((MN, D), F32),
            jax.ShapeDtypeStruct((MN, D), F32),
            jax.ShapeDtypeStruct((MC, D), F32),
            jax.ShapeDtypeStruct((MC, D), F32),
        ),
        mesh=mesh,
        scratch_types=[
            pltpu.VMEM((nw_n,), I32),
            pltpu.VMEM((nw_n,), I32),
            pltpu.VMEM((nw_c,), I32),
            pltpu.VMEM((nw_n, D), F32),
            pltpu.VMEM((nw_n, D), F32),
            pltpu.VMEM((nw_c, D), F32),
            pltpu.VMEM((nw_c, D), F32),
            pltpu.SemaphoreType.DMA,
            pltpu.SemaphoreType.DMA,
            pltpu.SemaphoreType.DMA,
            pltpu.SemaphoreType.DMA,
        ],
    )
    def k(xa_h, xb_h, hp_h, tr_h, ns_h, nt_h, tk_h,
          o1, o2, o3, o4, i1, i2, i3, r1, r2, r3, r4, s1, s2, s3, s4):
        wid = lax.axis_index("s") * NC + lax.axis_index("c")
        pltpu.sync_copy(ns_h.at[pl.ds(wid * nw_n, nw_n)], i1)
        pltpu.sync_copy(nt_h.at[pl.ds(wid * nw_n, nw_n)], i2)
        pltpu.sync_copy(tk_h.at[pl.ds(wid * nw_c, nw_c)], i3)
        c1 = pltpu.async_copy(xa_h.at[i1], r1, s1)
        c2 = pltpu.async_copy(xb_h.at[i2], r2, s2)
        c3 = pltpu.async_copy(hp_h.at[i3], r3, s3)
        c4 = pltpu.async_copy(tr_h.at[i3], r4, s4)
        c1.wait()
        c2.wait()
        c3.wait()
        c4.wait()
        pltpu.sync_copy(r1, o1.at[pl.ds(wid * nw_n, nw_n)])
        pltpu.sync_copy(r2, o2.at[pl.ds(wid * nw_n, nw_n)])
        pltpu.sync_copy(r3, o3.at[pl.ds(wid * nw_c, nw_c)])
        pltpu.sync_copy(r4, o4.at[pl.ds(wid * nw_c, nw_c)])

    return k(xa, xb, h_pre, trip, neg_src, neg_tail, topk_pad)


def _tail_body(hc_ref, ca_ref, ct_ref, gxa_ref, gxb_ref, na_ref, nt_ref,
               w1c_ref, relp_ref, w2_ref, xlc_ref, xln_ref,
               cz_ref, nz_ref, pos_ref, neg_ref):
    w1c = w1c_ref[...]
    relp = relp_ref[...]
    w2 = w2_ref[...]
    ohc = (lax.broadcasted_iota(I32, (256, RP), 1) == ct_ref[...]).astype(F32)
    bc = (jnp.dot(ca_ref[...], w1c, preferred_element_type=F32, precision=HI)
          + jnp.dot(ohc, relp, preferred_element_type=F32, precision=HI))
    cz = jnp.dot(jnp.maximum(hc_ref[...] + bc, 0.0), w2,
                 preferred_element_type=F32, precision=HI)
    cz_ref[...] = cz
    pos = jnp.sum(cz * xlc_ref[...], axis=1, keepdims=True)
    pos_ref[...] = jnp.broadcast_to(pos, (256, 128))
    ohn = (lax.broadcasted_iota(I32, (B * K * NEG, RP), 1)
           == nt_ref[...]).astype(F32)
    bn = (jnp.dot(na_ref[...], w1c, preferred_element_type=F32, precision=HI)
          + jnp.dot(ohn, relp, preferred_element_type=F32, precision=HI))
    hn = gxa_ref[...] + gxb_ref[...] + bn
    nz = jnp.dot(jnp.maximum(hn, 0.0), w2, preferred_element_type=F32,
                 precision=HI)
    nz_ref[...] = nz
    neg = jnp.sum(nz * xln_ref[...], axis=1, keepdims=True)
    neg_ref[...] = jnp.broadcast_to(neg, (B * K * NEG, 128))


def _tail(h_cand, cand_attr, cand_type, gxa, gxb, neg_attr, neg_type,
          W1c, relp, W2, xl_rep_c, xl_rep_n):
    MN = B * K * NEG
    return pl.pallas_call(
        _tail_body,
        out_shape=[
            jax.ShapeDtypeStruct((256, D), F32),
            jax.ShapeDtypeStruct((MN, D), F32),
            jax.ShapeDtypeStruct((256, 128), F32),
            jax.ShapeDtypeStruct((MN, 128), F32),
        ],
    )(h_cand, cand_attr, cand_type, gxa, gxb, neg_attr, neg_type,
      W1c, relp, W2, xl_rep_c, xl_rep_n)


def kernel(x_lm, x, edge_index, edge_type, edge_attr, triplet_embedding,
           neg_tail, W1, rel_emb, W2, k):
    del k
    src = edge_index[0]
    dst = edge_index[1]

    xa = x @ W1[:D]
    xb = x @ W1[D:2 * D]
    base = edge_attr @ W1[2 * D:]
    pad_e = E2 - E
    src_pad = jnp.concatenate([src, jnp.zeros((pad_e,), I32)])
    dst_pad = jnp.concatenate([dst, jnp.zeros((pad_e,), I32)])
    etype3 = edge_type.reshape(NBV, 1, EB)
    relp = jnp.concatenate([rel_emb, jnp.zeros((RP - R, D), F32)])
    W1c = W1[2 * D:, :]

    h_pre = _gather_add_all(xa, xb, src_pad, dst_pad)

    tki = _score_topk(h_pre, base, etype3, relp, x_lm, W2)
    topk_idx = tki[:, :K]

    flat_tk = topk_idx.reshape(B * K)
    topk_pad = jnp.concatenate([flat_tk, jnp.zeros((256 - B * K,), I32)])
    cand_src = jnp.take(src, flat_tk, axis=0)
    cand_type = jnp.take(edge_type, flat_tk, axis=0)
    cand_attr = jnp.take(edge_attr, flat_tk, axis=0)

    neg_src = jnp.repeat(cand_src.reshape(B, K), NEG, axis=1).reshape(-1)
    neg_type_f = jnp.repeat(cand_type.reshape(B, K), NEG, axis=1).reshape(-1)
    neg_attr = jnp.repeat(cand_attr.reshape(B, K, DE), NEG, axis=1
                          ).reshape(-1, DE)
    neg_tail_f = neg_tail.reshape(-1)

    gxa, gxb, h_cand, trip_cand = _gather_tail(
        xa, xb, h_pre, triplet_embedding, neg_src, neg_tail_f, topk_pad)

    cand_attr_p = jnp.concatenate(
        [cand_attr, jnp.zeros((256 - B * K, DE), F32)])
    cand_type_p = jnp.concatenate(
        [cand_type, jnp.zeros((256 - B * K,), I32)]).reshape(256, 1)
    xl_rep_c = jnp.concatenate(
        [jnp.repeat(x_lm, K, axis=0), jnp.zeros((256 - B * K, D), F32)])
    xl_rep_n = jnp.repeat(x_lm, K * NEG, axis=0)

    cz, nz, pos, neg = _tail(
        h_cand, cand_attr_p, cand_type_p, gxa, gxb,
        neg_attr, neg_type_f.reshape(B * K * NEG, 1), W1c, relp, W2,
        xl_rep_c, xl_rep_n)

    candidates_z = cz[:B * K].reshape(B, K, D)
    negatives_z = nz.reshape(B, K * NEG, D)
    candidates_lm_z = trip_cand[:B * K].reshape(B, K, D)
    pos_scores = pos[:B * K, 0].reshape(B, K)
    neg_scores = neg[:, 0].reshape(B, K * NEG)

    return (x_lm, pos_scores, neg_scores, candidates_z, candidates_lm_z,
            negatives_z)
